# baseline (speedup 1.0000x reference)
import jax
import jax.numpy as jnp
from jax import lax
from jax.experimental import pallas as pl
from jax.experimental.pallas import tpu as pltpu
from jax.experimental.pallas import tpu_sc as plsc

NC = 2
NS = 16
NW = NC * NS

B = 16384
DA = 128
DB = 64
BPW = B // NW
CH = 128
NCHUNK = BPW // CH


def _gather_body(idx_hbm, fa_hbm, fb_hbm, out_a, out_b,
                 idx_v, rows_a, rows_b, sem):
    wid = lax.axis_index("s") * NC + lax.axis_index("c")
    base = wid * BPW

    pltpu.sync_copy(idx_hbm.at[wid], idx_v)

    descs = []
    for j in range(NCHUNK):
        descs.append(pltpu.async_copy(
            fa_hbm.at[idx_v.at[j]], rows_a.at[pl.ds(j * CH, CH)], sem))
        descs.append(pltpu.async_copy(
            fb_hbm.at[idx_v.at[j]], rows_b.at[pl.ds(j * CH, CH)], sem))
    for d in descs:
        d.wait()

    pltpu.sync_copy(rows_a, out_a.at[pl.ds(base, BPW)])
    pltpu.sync_copy(rows_b, out_b.at[pl.ds(base, BPW)])


@jax.jit
def kernel(indices, feats_a, feats_b):
    idx = indices[:, 0].astype(jnp.int32).reshape(NW, NCHUNK, CH)
    mesh = plsc.VectorSubcoreMesh(core_axis_name="c", subcore_axis_name="s")
    out_a, out_b = pl.kernel(
        _gather_body,
        out_type=[
            jax.ShapeDtypeStruct((B, DA), jnp.float32),
            jax.ShapeDtypeStruct((B, DB), jnp.float32),
        ],
        mesh=mesh,
        compiler_params=pltpu.CompilerParams(use_tc_tiling_on_sc=False),
        scratch_types=[
            pltpu.VMEM((NCHUNK, CH), jnp.int32),
            pltpu.VMEM((BPW, DA), jnp.float32),
            pltpu.VMEM((BPW, DB), jnp.float32),
            pltpu.SemaphoreType.DMA,
        ],
    )(idx, feats_a, feats_b)
    return out_a, out_b
